# baseline (speedup 1.0000x reference)
import jax
import jax.numpy as jnp
from jax import lax
from jax.experimental import pallas as pl
from jax.experimental.pallas import tpu as pltpu
from jax.experimental.pallas import tpu_sc as plsc

N_NODES = 10000
N_EDGES = 320000
HID = 128

NC = 2
NS = 16
NW = NC * NS
CHUNK = 128

K_AGG = 80
E_PAD_A = NW * K_AGG * CHUNK

K_CNT = K_AGG

ROWS_PER_TILE = 640
NP = NS * ROWS_PER_TILE

_MESH = plsc.VectorSubcoreMesh(core_axis_name="c", subcore_axis_name="s")


def _fill(ref, value):
  @pl.loop(0, CHUNK)
  def _(i):
    @pl.loop(0, HID // 16)
    def _(j):
      ref[i, pl.ds(j * 16, 16)] = jnp.full((16,), value, jnp.float32)


def _agg_body(h_hbm, pk_hbm, p_hbm, acc, pk_v, rows0, rows1,
              si0, si1, di0, di1, sg0, sg1):
  cid = lax.axis_index("c")
  sid = lax.axis_index("s")
  wid = cid * NS + sid
  row0 = sid * ROWS_PER_TILE

  def unpack(c, src_sl, dst_sl):
    @pl.loop(0, CHUNK // 16)
    def _(j):
      v = pk_v[c, pl.ds(j * 16, 16)]
      src_sl[0, pl.ds(j * 16, 16)] = jnp.bitwise_and(v, 0xFFFF)
      dst_sl[0, pl.ds(j * 16, 16)] = jnp.right_shift(v, 16)

  pltpu.sync_copy(pk_hbm.at[wid], pk_v)

  _fill(rows0, 0.0)

  @pl.loop(0, ROWS_PER_TILE // CHUNK)
  def _(i):
    pltpu.sync_copy(rows0, acc.at[pl.ds(row0 + i * CHUNK, CHUNK)])

  plsc.subcore_barrier()

  @pl.loop(0, K_AGG)
  def _(c):
    unpack(c, si0, di0)
    pltpu.async_copy(h_hbm.at[si0.at[0]], rows0, sg0).wait()
    pltpu.sync_copy(rows0, acc.at[di0.at[0]], add=True)

  plsc.subcore_barrier()

  @pl.loop(0, ROWS_PER_TILE // CHUNK)
  def _(i):
    r = row0 + i * CHUNK
    pltpu.sync_copy(acc.at[pl.ds(r, CHUNK)], rows0)
    pltpu.sync_copy(rows0, p_hbm.at[cid, pl.ds(r, CHUNK)])


_sc_agg = pl.kernel(
    _agg_body,
    out_type=[jax.ShapeDtypeStruct((NC, NP, HID), jnp.float32)],
    mesh=_MESH,
    scratch_types=[
        pltpu.VMEM_SHARED((NP, HID), jnp.float32),
        pltpu.VMEM((K_AGG, CHUNK), jnp.int32),
        pltpu.VMEM((CHUNK, HID), jnp.float32),
        pltpu.VMEM((CHUNK, HID), jnp.float32),
        pltpu.VMEM((1, CHUNK), jnp.int32),
        pltpu.VMEM((1, CHUNK), jnp.int32),
        pltpu.VMEM((1, CHUNK), jnp.int32),
        pltpu.VMEM((1, CHUNK), jnp.int32),
        pltpu.SemaphoreType.DMA,
        pltpu.SemaphoreType.DMA,
    ],
)


def _cnt_body(pk_hbm, c_hbm, cacc, pk_v, di0, ones_v):
  cid = lax.axis_index("c")
  sid = lax.axis_index("s")
  wid = cid * NS + sid
  row0 = sid * ROWS_PER_TILE

  pltpu.sync_copy(pk_hbm.at[wid], pk_v)
  _fill(ones_v, 0.0)

  @pl.loop(0, ROWS_PER_TILE // CHUNK)
  def _(i):
    pltpu.sync_copy(ones_v, cacc.at[pl.ds(row0 + i * CHUNK, CHUNK)])

  _fill(ones_v, 1.0)

  plsc.subcore_barrier()

  @pl.loop(0, K_CNT)
  def _(c):
    @pl.loop(0, CHUNK // 16)
    def _(j):
      v = pk_v[c, pl.ds(j * 16, 16)]
      di0[0, pl.ds(j * 16, 16)] = jnp.right_shift(v, 16)
    pltpu.sync_copy(ones_v, cacc.at[di0.at[0]], add=True)

  plsc.subcore_barrier()

  @pl.loop(0, ROWS_PER_TILE // CHUNK)
  def _(i):
    r = row0 + i * CHUNK
    pltpu.sync_copy(cacc.at[pl.ds(r, CHUNK)], ones_v)
    pltpu.sync_copy(ones_v, c_hbm.at[cid, pl.ds(r, CHUNK)])


_sc_counts = pl.kernel(
    _cnt_body,
    out_type=[jax.ShapeDtypeStruct((NC, NP, HID), jnp.float32)],
    mesh=_MESH,
    scratch_types=[
        pltpu.VMEM_SHARED((NP, HID), jnp.float32),
        pltpu.VMEM((K_CNT, CHUNK), jnp.int32),
        pltpu.VMEM((1, CHUNK), jnp.int32),
        pltpu.VMEM((CHUNK, HID), jnp.float32),
    ],
)


ROW_BLK = ROWS_PER_TILE


def _enc_body(x_ref, w_ref, b_ref, o_ref):
  o_ref[...] = (
      lax.dot_general(x_ref[...], w_ref[...], (((1,), (1,)), ((), ())),
                      preferred_element_type=jnp.float32)
      + b_ref[...]
  )


def _encoder(x, w, b):
  return pl.pallas_call(
      _enc_body,
      grid=(NP // ROW_BLK,),
      in_specs=[
          pl.BlockSpec((ROW_BLK, HID), lambda i: (i, 0)),
          pl.BlockSpec((HID, HID), lambda i: (0, 0)),
          pl.BlockSpec((1, HID), lambda i: (0, 0)),
      ],
      out_specs=pl.BlockSpec((ROW_BLK, HID), lambda i: (i, 0)),
      out_shape=jax.ShapeDtypeStruct((NP, HID), jnp.float32),
  )(x, w, b.reshape(1, HID))


def _combine_common(p_ref, c_ref, h_ref, wl_ref, bl_ref, wr_ref):
  cnt = c_ref[0, :, 0:1] + c_ref[1, :, 0:1]
  recip = 1.0 / jnp.maximum(cnt, 1.0)
  aggr = (p_ref[0] + p_ref[1]) * recip
  return (
      lax.dot_general(aggr, wl_ref[...], (((1,), (1,)), ((), ())),
                      preferred_element_type=jnp.float32)
      + lax.dot_general(h_ref[...], wr_ref[...], (((1,), (1,)), ((), ())),
                        preferred_element_type=jnp.float32)
      + bl_ref[...]
  )


def _combine_body(p_ref, c_ref, h_ref, wl_ref, bl_ref, wr_ref, o_ref):
  o_ref[...] = _combine_common(p_ref, c_ref, h_ref, wl_ref, bl_ref, wr_ref)


def _combine(p, c, h, wl, bl, wr, n_rows, blk):
  return pl.pallas_call(
      _combine_body,
      grid=(n_rows // blk,),
      in_specs=[
          pl.BlockSpec((NC, blk, HID), lambda i: (0, i, 0)),
          pl.BlockSpec((NC, blk, HID), lambda i: (0, i, 0)),
          pl.BlockSpec((blk, HID), lambda i: (i, 0)),
          pl.BlockSpec((HID, HID), lambda i: (0, 0)),
          pl.BlockSpec((1, HID), lambda i: (0, 0)),
          pl.BlockSpec((HID, HID), lambda i: (0, 0)),
      ],
      out_specs=pl.BlockSpec((blk, HID), lambda i: (i, 0)),
      out_shape=jax.ShapeDtypeStruct((n_rows, HID), jnp.float32),
  )(p, c, h, wl, bl.reshape(1, HID), wr)


@jax.jit
def kernel(g, x, W_enc, b_enc, Wl0, bl0, Wr0, Wl1, bl1, Wr1):
  src = g[0].astype(jnp.int32)
  dst = g[1].astype(jnp.int32)
  pad = E_PAD_A - N_EDGES
  dummy_dst = N_NODES + jnp.arange(pad, dtype=jnp.int32) % (NP - N_NODES)
  pk = src + dst * 65536
  pk_a = jnp.concatenate([pk, dummy_dst * 65536]) \
      .reshape(K_AGG, NW, CHUNK).transpose(1, 0, 2)

  x_pad = jnp.pad(x, ((0, NP - N_NODES), (0, 0)))

  h0 = _encoder(x_pad, W_enc, b_enc)
  (c,) = _sc_counts(pk_a)
  (p1,) = _sc_agg(h0, pk_a)
  h1 = _combine(p1, c, h0, Wl0, bl0, Wr0, NP, ROW_BLK)
  (p2,) = _sc_agg(h1, pk_a)
  h2 = _combine(p2, c, h1, Wl1, bl1, Wr1, N_NODES, 400)
  return h2


# speedup vs baseline: 1.3206x; 1.3206x over previous
import jax
import jax.numpy as jnp
from jax import lax
from jax.experimental import pallas as pl
from jax.experimental.pallas import tpu as pltpu
from jax.experimental.pallas import tpu_sc as plsc

N_NODES = 10000
N_EDGES = 320000
HID = 128

NC = 2
NS = 16
NW = NC * NS
CHUNK = 128
K_CHUNKS = (N_EDGES + NW * CHUNK - 1) // (NW * CHUNK)
E_PAD = NW * K_CHUNKS * CHUNK
ROWS_PER_TILE = 640
N_PAD = NS * ROWS_PER_TILE

ROW_BLK = 400

_MESH = plsc.VectorSubcoreMesh(core_axis_name="c", subcore_axis_name="s")


def _fill(ref, value):
  @pl.loop(0, CHUNK)
  def _(i):
    @pl.loop(0, HID // 16)
    def _(j):
      ref[i, pl.ds(j * 16, 16)] = jnp.full((16,), value, jnp.float32)


def _agg_body(h_hbm, src_hbm, dst_hbm, p_hbm, acc, src_v, dst_v, rows_v, sem):
  cid = lax.axis_index("c")
  sid = lax.axis_index("s")
  wid = cid * NS + sid
  row0 = sid * ROWS_PER_TILE

  _fill(rows_v, 0.0)

  @pl.loop(0, ROWS_PER_TILE // CHUNK)
  def _(i):
    pltpu.sync_copy(rows_v, acc.at[pl.ds(row0 + i * CHUNK, CHUNK)])

  pltpu.sync_copy(src_hbm.at[wid], src_v)
  pltpu.sync_copy(dst_hbm.at[wid], dst_v)

  plsc.subcore_barrier()

  @pl.loop(0, K_CHUNKS)
  def _(j):
    pltpu.async_copy(h_hbm.at[src_v.at[j]], rows_v, sem).wait()
    pltpu.sync_copy(rows_v, acc.at[dst_v.at[j]], add=True)

  plsc.subcore_barrier()

  @pl.loop(0, ROWS_PER_TILE // CHUNK)
  def _(i):
    r = row0 + i * CHUNK
    pltpu.sync_copy(acc.at[pl.ds(r, CHUNK)], rows_v)
    pltpu.sync_copy(rows_v, p_hbm.at[cid, pl.ds(r, CHUNK)])


_sc_agg = pl.kernel(
    _agg_body,
    out_type=[jax.ShapeDtypeStruct((NC, N_PAD, HID), jnp.float32)],
    mesh=_MESH,
    scratch_types=[
        pltpu.VMEM_SHARED((N_PAD, HID), jnp.float32),
        pltpu.VMEM((K_CHUNKS, CHUNK), jnp.int32),
        pltpu.VMEM((K_CHUNKS, CHUNK), jnp.int32),
        pltpu.VMEM((CHUNK, HID), jnp.float32),
        pltpu.SemaphoreType.DMA,
    ],
)


def _cnt_body(dst_hbm, c_hbm, cacc, dst_v, const_v):
  cid = lax.axis_index("c")
  sid = lax.axis_index("s")
  wid = cid * NS + sid
  row0 = sid * ROWS_PER_TILE

  _fill(const_v, 0.0)

  @pl.loop(0, ROWS_PER_TILE // CHUNK)
  def _(i):
    pltpu.sync_copy(const_v, cacc.at[pl.ds(row0 + i * CHUNK, CHUNK)])

  pltpu.sync_copy(dst_hbm.at[wid], dst_v)

  _fill(const_v, 1.0)

  plsc.subcore_barrier()

  @pl.loop(0, K_CHUNKS)
  def _(j):
    pltpu.sync_copy(const_v, cacc.at[dst_v.at[j]], add=True)

  plsc.subcore_barrier()

  @pl.loop(0, ROWS_PER_TILE // CHUNK)
  def _(i):
    r = row0 + i * CHUNK
    pltpu.sync_copy(cacc.at[pl.ds(r, CHUNK)], const_v)
    pltpu.sync_copy(const_v, c_hbm.at[cid, pl.ds(r, CHUNK)])


_sc_counts = pl.kernel(
    _cnt_body,
    out_type=[jax.ShapeDtypeStruct((NC, N_PAD, HID), jnp.float32)],
    mesh=_MESH,
    scratch_types=[
        pltpu.VMEM_SHARED((N_PAD, HID), jnp.float32),
        pltpu.VMEM((K_CHUNKS, CHUNK), jnp.int32),
        pltpu.VMEM((CHUNK, HID), jnp.float32),
    ],
)


def _enc_body(x_ref, w_ref, b_ref, o_ref):
  o_ref[...] = (
      lax.dot_general(x_ref[...], w_ref[...], (((1,), (1,)), ((), ())),
                      preferred_element_type=jnp.float32)
      + b_ref[...]
  )


def _encoder(x, w, b):
  return pl.pallas_call(
      _enc_body,
      grid=(N_NODES // ROW_BLK,),
      in_specs=[
          pl.BlockSpec((ROW_BLK, HID), lambda i: (i, 0)),
          pl.BlockSpec((HID, HID), lambda i: (0, 0)),
          pl.BlockSpec((1, HID), lambda i: (0, 0)),
      ],
      out_specs=pl.BlockSpec((ROW_BLK, HID), lambda i: (i, 0)),
      out_shape=jax.ShapeDtypeStruct((N_NODES, HID), jnp.float32),
  )(x, w, b.reshape(1, HID))


def _combine_body(p_ref, c_ref, h_ref, wl_ref, bl_ref, wr_ref, o_ref):
  cnt = c_ref[0, :, 0:1] + c_ref[1, :, 0:1]
  recip = 1.0 / jnp.maximum(cnt, 1.0)
  aggr = (p_ref[0] + p_ref[1]) * recip
  o_ref[...] = (
      lax.dot_general(aggr, wl_ref[...], (((1,), (1,)), ((), ())),
                      preferred_element_type=jnp.float32)
      + lax.dot_general(h_ref[...], wr_ref[...], (((1,), (1,)), ((), ())),
                        preferred_element_type=jnp.float32)
      + bl_ref[...]
  )


def _combine(p, c, h, wl, bl, wr):
  return pl.pallas_call(
      _combine_body,
      grid=(N_NODES // ROW_BLK,),
      in_specs=[
          pl.BlockSpec((NC, ROW_BLK, HID), lambda i: (0, i, 0)),
          pl.BlockSpec((NC, ROW_BLK, HID), lambda i: (0, i, 0)),
          pl.BlockSpec((ROW_BLK, HID), lambda i: (i, 0)),
          pl.BlockSpec((HID, HID), lambda i: (0, 0)),
          pl.BlockSpec((1, HID), lambda i: (0, 0)),
          pl.BlockSpec((HID, HID), lambda i: (0, 0)),
      ],
      out_specs=pl.BlockSpec((ROW_BLK, HID), lambda i: (i, 0)),
      out_shape=jax.ShapeDtypeStruct((N_NODES, HID), jnp.float32),
  )(p, c, h, wl, bl.reshape(1, HID), wr)


@jax.jit
def kernel(g, x, W_enc, b_enc, Wl0, bl0, Wr0, Wl1, bl1, Wr1):
  src = g[0].astype(jnp.int32)
  dst = g[1].astype(jnp.int32)
  pad = E_PAD - N_EDGES
  src_p = jnp.concatenate([src, jnp.zeros((pad,), jnp.int32)]).reshape(
      NW, K_CHUNKS, CHUNK)
  dst_p = jnp.concatenate(
      [dst, jnp.full((pad,), N_NODES, jnp.int32)]).reshape(
      NW, K_CHUNKS, CHUNK)

  h0 = _encoder(x, W_enc, b_enc)
  (c,) = _sc_counts(dst_p)
  (p1,) = _sc_agg(h0, src_p, dst_p)
  h1 = _combine(p1, c, h0, Wl0, bl0, Wr0)
  (p2,) = _sc_agg(h1, src_p, dst_p)
  h2 = _combine(p2, c, h1, Wl1, bl1, Wr1)
  return h2


# speedup vs baseline: 1.4070x; 1.0654x over previous
import jax
import jax.numpy as jnp
from jax import lax
from jax.experimental import pallas as pl
from jax.experimental.pallas import tpu as pltpu
from jax.experimental.pallas import tpu_sc as plsc

N_NODES = 10000
N_EDGES = 320000
HID = 128

NC = 2
NS = 16
NW = NC * NS
CHUNK = 128
K_CHUNKS = (N_EDGES + NW * CHUNK - 1) // (NW * CHUNK)
E_PAD = NW * K_CHUNKS * CHUNK
ROWS_PER_TILE = 640
N_PAD = NS * ROWS_PER_TILE

ROW_BLK = 400

_MESH = plsc.VectorSubcoreMesh(core_axis_name="c", subcore_axis_name="s")


def _fill(ref, value):
  @pl.loop(0, CHUNK)
  def _(i):
    @pl.loop(0, HID // 16)
    def _(j):
      ref[i, pl.ds(j * 16, 16)] = jnp.full((16,), value, jnp.float32)


def _agg_body(h_hbm, src_hbm, dst_hbm, p_hbm, acc, src_v, dst_v, rows_v, sem):
  cid = lax.axis_index("c")
  sid = lax.axis_index("s")
  wid = cid * NS + sid
  row0 = sid * ROWS_PER_TILE

  _fill(rows_v, 0.0)

  @pl.loop(0, ROWS_PER_TILE // CHUNK)
  def _(i):
    pltpu.sync_copy(rows_v, acc.at[pl.ds(row0 + i * CHUNK, CHUNK)])

  pltpu.sync_copy(src_hbm.at[wid], src_v)
  pltpu.sync_copy(dst_hbm.at[wid], dst_v)

  plsc.subcore_barrier()

  @pl.loop(0, K_CHUNKS)
  def _(j):
    pltpu.async_copy(h_hbm.at[src_v.at[j]], rows_v, sem).wait()
    pltpu.sync_copy(rows_v, acc.at[dst_v.at[j]], add=True)

  plsc.subcore_barrier()

  @pl.loop(0, ROWS_PER_TILE // CHUNK)
  def _(i):
    r = row0 + i * CHUNK
    pltpu.sync_copy(acc.at[pl.ds(r, CHUNK)], rows_v)
    pltpu.sync_copy(rows_v, p_hbm.at[cid, pl.ds(r, CHUNK)])


_sc_agg = pl.kernel(
    _agg_body,
    out_type=[jax.ShapeDtypeStruct((NC, N_PAD, HID), jnp.float32)],
    mesh=_MESH,
    scratch_types=[
        pltpu.VMEM_SHARED((N_PAD, HID), jnp.float32),
        pltpu.VMEM((K_CHUNKS, CHUNK), jnp.int32),
        pltpu.VMEM((K_CHUNKS, CHUNK), jnp.int32),
        pltpu.VMEM((CHUNK, HID), jnp.float32),
        pltpu.SemaphoreType.DMA,
    ],
)


def _cnt_body(dst_hbm, c_hbm, cacc, dst_v, const_v):
  cid = lax.axis_index("c")
  sid = lax.axis_index("s")
  wid = cid * NS + sid
  row0 = sid * ROWS_PER_TILE

  _fill(const_v, 0.0)

  @pl.loop(0, ROWS_PER_TILE // CHUNK)
  def _(i):
    pltpu.sync_copy(const_v, cacc.at[pl.ds(row0 + i * CHUNK, CHUNK)])

  pltpu.sync_copy(dst_hbm.at[wid], dst_v)

  _fill(const_v, 1.0)

  plsc.subcore_barrier()

  @pl.loop(0, K_CHUNKS)
  def _(j):
    pltpu.sync_copy(const_v, cacc.at[dst_v.at[j]], add=True)

  plsc.subcore_barrier()

  @pl.loop(0, ROWS_PER_TILE // CHUNK)
  def _(i):
    r = row0 + i * CHUNK
    pltpu.sync_copy(cacc.at[pl.ds(r, CHUNK)], const_v)
    pltpu.sync_copy(const_v, c_hbm.at[cid, pl.ds(r, CHUNK)])


_sc_counts = pl.kernel(
    _cnt_body,
    out_type=[jax.ShapeDtypeStruct((NC, N_PAD, HID), jnp.float32)],
    mesh=_MESH,
    scratch_types=[
        pltpu.VMEM_SHARED((N_PAD, HID), jnp.float32),
        pltpu.VMEM((K_CHUNKS, CHUNK), jnp.int32),
        pltpu.VMEM((CHUNK, HID), jnp.float32),
    ],
)


def _enc_body(x_ref, w_ref, b_ref, o_ref):
  o_ref[...] = (
      lax.dot_general(x_ref[...], w_ref[...], (((1,), (1,)), ((), ())),
                      preferred_element_type=jnp.float32)
      + b_ref[...]
  )


def _encoder(x, w, b):
  return pl.pallas_call(
      _enc_body,
      grid=(N_NODES // ROW_BLK,),
      in_specs=[
          pl.BlockSpec((ROW_BLK, HID), lambda i: (i, 0)),
          pl.BlockSpec((HID, HID), lambda i: (0, 0)),
          pl.BlockSpec((1, HID), lambda i: (0, 0)),
      ],
      out_specs=pl.BlockSpec((ROW_BLK, HID), lambda i: (i, 0)),
      out_shape=jax.ShapeDtypeStruct((N_NODES, HID), jnp.float32),
  )(x, w, b.reshape(1, HID))


def _combine_body(p_ref, c_ref, h_ref, wl_ref, bl_ref, wr_ref, o_ref):
  cnt = c_ref[0, :, 0:1] + c_ref[1, :, 0:1]
  recip = 1.0 / jnp.maximum(cnt, 1.0)
  aggr = (p_ref[0] + p_ref[1]) * recip
  o_ref[...] = (
      lax.dot_general(aggr, wl_ref[...], (((1,), (1,)), ((), ())),
                      preferred_element_type=jnp.float32)
      + lax.dot_general(h_ref[...], wr_ref[...], (((1,), (1,)), ((), ())),
                        preferred_element_type=jnp.float32)
      + bl_ref[...]
  )


def _combine(p, c, h, wl, bl, wr):
  return pl.pallas_call(
      _combine_body,
      grid=(N_NODES // ROW_BLK,),
      in_specs=[
          pl.BlockSpec((NC, ROW_BLK, HID), lambda i: (0, i, 0)),
          pl.BlockSpec((NC, ROW_BLK, HID), lambda i: (0, i, 0)),
          pl.BlockSpec((ROW_BLK, HID), lambda i: (i, 0)),
          pl.BlockSpec((HID, HID), lambda i: (0, 0)),
          pl.BlockSpec((1, HID), lambda i: (0, 0)),
          pl.BlockSpec((HID, HID), lambda i: (0, 0)),
      ],
      out_specs=pl.BlockSpec((ROW_BLK, HID), lambda i: (i, 0)),
      out_shape=jax.ShapeDtypeStruct((N_NODES, HID), jnp.float32),
  )(p, c, h, wl, bl.reshape(1, HID), wr)


@jax.jit
def kernel(g, x, W_enc, b_enc, Wl0, bl0, Wr0, Wl1, bl1, Wr1):
  src = g[0].astype(jnp.int32)
  dst = g[1].astype(jnp.int32)
  pad = E_PAD - N_EDGES
  dummy_dst = N_NODES + jnp.arange(pad, dtype=jnp.int32) % (N_PAD - N_NODES)
  src_p = jnp.concatenate([src, jnp.zeros((pad,), jnp.int32)]) \
      .reshape(K_CHUNKS, NW, CHUNK).transpose(1, 0, 2)
  dst_p = jnp.concatenate([dst, dummy_dst]) \
      .reshape(K_CHUNKS, NW, CHUNK).transpose(1, 0, 2)
  h0 = _encoder(x, W_enc, b_enc)
  (c,) = _sc_counts(dst_p)
  (p1,) = _sc_agg(h0, src_p, dst_p)
  h1 = _combine(p1, c, h0, Wl0, bl0, Wr0)
  (p2,) = _sc_agg(h1, src_p, dst_p)
  h2 = _combine(p2, c, h1, Wl1, bl1, Wr1)
  return h2
